# baseline (speedup 1.0000x reference)
import functools

import jax
import jax.numpy as jnp
from jax import lax
from jax.experimental import pallas as pl
from jax.experimental.pallas import tpu as pltpu
from jax.experimental.pallas import tpu_sc as plsc

_N = 10000
_E = 320000
_D = 128
_DH = _D // 2
_NC = 2
_NS = 16
_NW = _NC * _NS
_CW = 128
_CPT_DEG = 80
_CPT_AGG = 160
_EPAD = _NW * _CPT_DEG * _CW
_NPAD = 10240
_RPT = _NPAD // _NS
_R = 1000
_G = _N // _R

_f32 = jnp.float32


def _sc_mesh():
    return plsc.VectorSubcoreMesh(
        core_axis_name="c", subcore_axis_name="s", num_cores=_NC,
        num_subcores=_NS)


def _fill(rows_v, value):
    v = jnp.full((16,), value, _f32)

    def body(i, carry):
        for k in range(_DH // 16):
            rows_v[i, pl.ds(k * 16, 16)] = v
        return carry

    lax.fori_loop(0, _CW, body, 0)


def _init_acc(s, rows_v, acc_sh):
    _fill(rows_v, 0.0)
    for t in range(_RPT // _CW):
        pltpu.sync_copy(rows_v, acc_sh.at[pl.ds(s * _RPT + t * _CW, _CW)])


def _deg_body(dsts_hbm, out_hbm, dst_v, rows_v, acc_sh, sem):
    c = lax.axis_index("c")
    s = lax.axis_index("s")
    wid = c * _NS + s
    pltpu.sync_copy(dsts_hbm.at[pl.ds(wid * _CPT_DEG, _CPT_DEG)], dst_v)
    _init_acc(s, rows_v, acc_sh)
    plsc.subcore_barrier()
    _fill(rows_v, 1.0)

    def step(j, carry):
        pltpu.sync_copy(rows_v, acc_sh.at[dst_v.at[j]], add=True)
        return carry

    lax.fori_loop(0, _CPT_DEG, step, 0)
    plsc.subcore_barrier()
    pltpu.sync_copy(acc_sh.at[pl.ds(s * _RPT, _RPT)],
                    out_hbm.at[c, pl.ds(s * _RPT, _RPT)])


_RING = 4
_SLAB = 16
_NSLAB = _CPT_AGG // _SLAB
_HPT = _N // _NS


def _agg_body(h_hbm, srcs_hbm, dsts_hbm, out_hbm, sv0, sv1, dv0, dv1,
              r0, r1, r2, r3, h_sh, acc_sh, lsem0, lsem1,
              g0, g1, g2, g3, s0, s1, s2, s3):
    rows = (r0, r1, r2, r3)
    gsem = (g0, g1, g2, g3)
    ssem = (s0, s1, s2, s3)
    src_sl = (sv0, sv1)
    dst_sl = (dv0, dv1)
    lsem = (lsem0, lsem1)
    c = lax.axis_index("c")
    s = lax.axis_index("s")
    base = s * _CPT_AGG

    pltpu.sync_copy(h_hbm.at[c, pl.ds(s * _HPT, _HPT)],
                    h_sh.at[pl.ds(s * _HPT, _HPT)])
    _init_acc(s, rows[0], acc_sh)

    def load_slab(q, qb):
        pltpu.async_copy(srcs_hbm.at[pl.ds(base + q * _SLAB, _SLAB)],
                         src_sl[qb], lsem[qb])
        pltpu.async_copy(dsts_hbm.at[pl.ds(base + q * _SLAB, _SLAB)],
                         dst_sl[qb], lsem[qb])

    def wait_slab(q, qb):
        pltpu.make_async_copy(srcs_hbm.at[pl.ds(base + q * _SLAB, _SLAB)],
                              src_sl[qb], lsem[qb]).wait()
        pltpu.make_async_copy(dsts_hbm.at[pl.ds(base + q * _SLAB, _SLAB)],
                              dst_sl[qb], lsem[qb]).wait()

    def gather(t, qb, b):
        pltpu.async_copy(h_sh.at[src_sl[qb].at[t]], rows[b], gsem[b])

    def wait_gather(t, qb, b):
        pltpu.make_async_copy(h_sh.at[src_sl[qb].at[t]], rows[b],
                              gsem[b]).wait()

    def scatter(t, qb, b):
        pltpu.async_copy(rows[b], acc_sh.at[dst_sl[qb].at[t]], ssem[b],
                         add=True)

    def wait_scatter(t, qb, b):
        pltpu.make_async_copy(rows[b], acc_sh.at[dst_sl[qb].at[t]],
                              ssem[b]).wait()

    load_slab(0, 0)
    wait_slab(0, 0)
    plsc.subcore_barrier()
    gather(0, 0, 0)
    gather(1, 0, 1)

    def pairbody(q2, carry):
        for qq in range(2):
            qb = qq
            qn = 1 - qq
            q = q2 * 2 + qq
            for t in range(_SLAB):
                jj = q * _SLAB + t
                b = t % _RING
                wait_gather(t, qb, b)
                scatter(t, qb, b)
                if t == 2:
                    @pl.when(q + 1 < _NSLAB)
                    def _():
                        load_slab(q + 1, qn)
                if t == 13:
                    @pl.when(q + 1 < _NSLAB)
                    def _():
                        wait_slab(q + 1, qn)
                bn = (t + 2) % _RING

                @pl.when(jj >= 2)
                def _():
                    if t >= 2:
                        wait_scatter(t - 2, qb, bn)
                    else:
                        wait_scatter(t + _SLAB - 2, qn, bn)

                @pl.when(jj + 2 < _CPT_AGG)
                def _():
                    if t < _SLAB - 2:
                        gather(t + 2, qb, bn)
                    else:
                        gather(t + 2 - _SLAB, qn, bn)
        return carry

    lax.fori_loop(0, _NSLAB // 2, pairbody, 0)
    for t in (_SLAB - 2, _SLAB - 1):
        wait_scatter(t, (_NSLAB - 1) % 2, t % _RING)
    plsc.subcore_barrier()
    pltpu.sync_copy(acc_sh.at[pl.ds(s * _RPT, _RPT)],
                    out_hbm.at[c, pl.ds(s * _RPT, _RPT)])


def _sc_deg(dsts):
    fn = pl.kernel(
        _deg_body,
        out_type=jax.ShapeDtypeStruct((_NC, _NPAD, _DH), _f32),
        mesh=_sc_mesh(),
        compiler_params=pltpu.CompilerParams(use_tc_tiling_on_sc=False),
        scratch_types=[
            pltpu.VMEM((_CPT_DEG, _CW), jnp.int32),
            pltpu.VMEM((_CW, _DH), _f32),
            pltpu.VMEM_SHARED((_NPAD, _DH), _f32),
            pltpu.SemaphoreType.DMA,
        ],
    )
    return fn(dsts)


def _sc_agg(h, srcs, dsts):
    fn = pl.kernel(
        _agg_body,
        out_type=jax.ShapeDtypeStruct((_NC, _NPAD, _DH), _f32),
        mesh=_sc_mesh(),
        compiler_params=pltpu.CompilerParams(use_tc_tiling_on_sc=False),
        scratch_types=(
            [pltpu.VMEM((_SLAB, _CW), jnp.int32)] * 4
            + [pltpu.VMEM((_CW, _DH), _f32)] * _RING
            + [pltpu.VMEM_SHARED((_N, _DH), _f32)]
            + [pltpu.VMEM_SHARED((_NPAD, _DH), _f32)]
            + [pltpu.SemaphoreType.DMA] * (2 + 2 * _RING)
        ),
    )
    return fn(h, srcs, dsts)


def _row_spec():
    return pl.BlockSpec((_R, _D), lambda i: (i, 0))


def _half_spec():
    return pl.BlockSpec((_NC, _R, _DH), lambda i: (0, i, 0))


def _full_spec(shape):
    return pl.BlockSpec(shape, lambda i: tuple(0 for _ in shape))


def _split_store(out_ref, val):
    out_ref[0, :, :] = val[:, :_DH]
    out_ref[1, :, :] = val[:, _DH:]


def _cat(ref):
    return jnp.concatenate([ref[0], ref[1]], axis=-1)


def _mm_body(x_ref, w_ref, y_ref):
    y_ref[...] = jnp.dot(x_ref[...], w_ref[...],
                         preferred_element_type=_f32)


def _tc_mm(x, w):
    return pl.pallas_call(
        _mm_body,
        grid=(_G,),
        in_specs=[_row_spec(), _full_spec((_D, _D))],
        out_specs=_row_spec(),
        out_shape=jax.ShapeDtypeStruct((_N, _D), _f32),
    )(x, w)


def _scale1_body(deg_ref, y_ref, dinv_ref, h_ref):
    deg64 = deg_ref[0] + deg_ref[1] + 1.0
    dinv = jnp.concatenate([lax.rsqrt(deg64)] * 2, axis=-1)
    dinv_ref[...] = dinv
    _split_store(h_ref, dinv * y_ref[...])


def _tc_layer1(deg, y, w=None):
    return pl.pallas_call(
        _scale1_body,
        grid=(_G,),
        in_specs=[_half_spec(), _row_spec()],
        out_specs=[_row_spec(), _half_spec()],
        out_shape=[
            jax.ShapeDtypeStruct((_N, _D), _f32),
            jax.ShapeDtypeStruct((_NC, _N, _DH), _f32),
        ],
    )(deg, y)


def _layer_body(agg_ref, hp_ref, dinv_ref, b_ref, w_ref, out_ref):
    dinv = dinv_ref[...]
    a = dinv * (_cat(agg_ref) + _cat(hp_ref)) + b_ref[...]
    a = jnp.maximum(a, 0.0)
    y = jnp.dot(a, w_ref[...], preferred_element_type=_f32)
    _split_store(out_ref, dinv * y)


def _tc_layer(agg, hp, dinv, b, w):
    return pl.pallas_call(
        _layer_body,
        grid=(_G,),
        in_specs=[_half_spec(), _half_spec(), _row_spec(),
                  _full_spec((1, _D)), _full_spec((_D, _D))],
        out_specs=_half_spec(),
        out_shape=jax.ShapeDtypeStruct((_NC, _N, _DH), _f32),
    )(agg, hp, dinv, b, w)


def _final_body(agg_ref, hp_ref, dinv_ref, b_ref, out_ref):
    out_ref[...] = (dinv_ref[...] * (_cat(agg_ref) + _cat(hp_ref))
                    + b_ref[...])


def _tc_final(agg, hp, dinv, b):
    return pl.pallas_call(
        _final_body,
        grid=(_G,),
        in_specs=[_half_spec(), _half_spec(), _row_spec(),
                  _full_spec((1, _D))],
        out_specs=_row_spec(),
        out_shape=jax.ShapeDtypeStruct((_N, _D), _f32),
    )(agg, hp, dinv, b)


def kernel(x, edge_index, W1, b1, W2, b2, W3, b3):
    src = edge_index[0]
    dst = edge_index[1]
    pad = _EPAD - _E
    srcs = jnp.concatenate(
        [src, jnp.zeros((pad,), jnp.int32)]).reshape(_EPAD // _CW, _CW)
    dsts = jnp.concatenate(
        [dst, jnp.full((pad,), _N, jnp.int32)]).reshape(_EPAD // _CW, _CW)
    b1r = b1.reshape(1, _D)
    b2r = b2.reshape(1, _D)
    b3r = b3.reshape(1, _D)

    y1 = _tc_mm(x, W1)
    deg = _sc_deg(dsts)
    dinv, h1 = _tc_layer1(deg[:, :_N], y1)
    agg1 = _sc_agg(h1, srcs, dsts)
    h2 = _tc_layer(agg1[:, :_N], h1, dinv, b1r, W2)
    agg2 = _sc_agg(h2, srcs, dsts)
    h3 = _tc_layer(agg2[:, :_N], h2, dinv, b2r, W3)
    agg3 = _sc_agg(h3, srcs, dsts)
    return _tc_final(agg3[:, :_N], h3, dinv, b3r)


# speedup vs baseline: 1.0425x; 1.0425x over previous
import functools

import jax
import jax.numpy as jnp
from jax import lax
from jax.experimental import pallas as pl
from jax.experimental.pallas import tpu as pltpu
from jax.experimental.pallas import tpu_sc as plsc

_N = 10000
_E = 320000
_D = 128
_DH = _D // 2
_NC = 2
_NS = 16
_NW = _NC * _NS
_CW = 128
_CPT_DEG = 80
_CPT_AGG = 160
_EPAD = _NW * _CPT_DEG * _CW
_NPAD = 10240
_RPT = _NPAD // _NS
_R = 1000
_G = _N // _R

_f32 = jnp.float32


def _sc_mesh():
    return plsc.VectorSubcoreMesh(
        core_axis_name="c", subcore_axis_name="s", num_cores=_NC,
        num_subcores=_NS)


def _fill(rows_v, value):
    v = jnp.full((16,), value, _f32)

    def body(i, carry):
        for k in range(_DH // 16):
            rows_v[i, pl.ds(k * 16, 16)] = v
        return carry

    lax.fori_loop(0, _CW, body, 0)


def _init_acc(s, rows_v, acc_sh):
    _fill(rows_v, 0.0)
    for t in range(_RPT // _CW):
        pltpu.sync_copy(rows_v, acc_sh.at[pl.ds(s * _RPT + t * _CW, _CW)])


_HR = _NPAD // 16
_HRT = _HR // _NS


def _deg_body(dsts_hbm, out_hbm, dst_v, hist_v, idv, acc_sh):
    c = lax.axis_index("c")
    s = lax.axis_index("s")
    wid = c * _NS + s
    pltpu.sync_copy(dsts_hbm.at[pl.ds(wid * _CPT_DEG, _CPT_DEG)], dst_v)
    z16 = jnp.zeros((16,), _f32)

    def zrow(r, carry):
        hist_v[r] = z16
        return carry

    lax.fori_loop(0, _HR, zrow, 0)
    iota = lax.iota(jnp.int32, 16)
    for p in range(_HR // _CW):
        for k in range(8):
            idv[p, pl.ds(k * 16, 16)] = iota + (p * _CW + k * 16)
    pltpu.sync_copy(hist_v.at[pl.ds(0, _HRT)],
                    acc_sh.at[pl.ds(s * _HRT, _HRT)])
    plsc.subcore_barrier()

    ones = jnp.ones((16,), _f32)

    def step(j, carry):
        for k in range(8):
            idx = dst_v[j, pl.ds(k * 16, 16)]
            plsc.addupdate_scatter(
                hist_v, [lax.shift_right_logical(idx, 4),
                         lax.bitwise_and(idx, 15)], ones)
        return carry

    lax.fori_loop(0, _CPT_DEG, step, 0)
    for p in range(_HR // _CW):
        pltpu.sync_copy(hist_v.at[pl.ds(p * _CW, _CW)],
                        acc_sh.at[idv.at[p]], add=True)
    plsc.subcore_barrier()
    pltpu.sync_copy(acc_sh.at[pl.ds(s * _HRT, _HRT)],
                    out_hbm.at[c, pl.ds(s * _HRT, _HRT)])


_RING = 4
_SLAB = 16
_NSLAB = _CPT_AGG // _SLAB
_HPT = _N // _NS


def _agg_body(h_hbm, srcs_hbm, dsts_hbm, out_hbm, sv0, sv1, dv0, dv1,
              r0, r1, r2, r3, h_sh, acc_sh, lsem0, lsem1,
              g0, g1, g2, g3, s0, s1, s2, s3):
    rows = (r0, r1, r2, r3)
    gsem = (g0, g1, g2, g3)
    ssem = (s0, s1, s2, s3)
    src_sl = (sv0, sv1)
    dst_sl = (dv0, dv1)
    lsem = (lsem0, lsem1)
    c = lax.axis_index("c")
    s = lax.axis_index("s")
    base = s * _CPT_AGG

    pltpu.sync_copy(h_hbm.at[c, pl.ds(s * _HPT, _HPT)],
                    h_sh.at[pl.ds(s * _HPT, _HPT)])
    _init_acc(s, rows[0], acc_sh)

    def load_slab(q, qb):
        pltpu.async_copy(srcs_hbm.at[pl.ds(base + q * _SLAB, _SLAB)],
                         src_sl[qb], lsem[qb])
        pltpu.async_copy(dsts_hbm.at[pl.ds(base + q * _SLAB, _SLAB)],
                         dst_sl[qb], lsem[qb])

    def wait_slab(q, qb):
        pltpu.make_async_copy(srcs_hbm.at[pl.ds(base + q * _SLAB, _SLAB)],
                              src_sl[qb], lsem[qb]).wait()
        pltpu.make_async_copy(dsts_hbm.at[pl.ds(base + q * _SLAB, _SLAB)],
                              dst_sl[qb], lsem[qb]).wait()

    def gather(t, qb, b):
        pltpu.async_copy(h_sh.at[src_sl[qb].at[t]], rows[b], gsem[b])

    def wait_gather(t, qb, b):
        pltpu.make_async_copy(h_sh.at[src_sl[qb].at[t]], rows[b],
                              gsem[b]).wait()

    def scatter(t, qb, b):
        pltpu.async_copy(rows[b], acc_sh.at[dst_sl[qb].at[t]], ssem[b],
                         add=True)

    def wait_scatter(t, qb, b):
        pltpu.make_async_copy(rows[b], acc_sh.at[dst_sl[qb].at[t]],
                              ssem[b]).wait()

    load_slab(0, 0)
    wait_slab(0, 0)
    plsc.subcore_barrier()
    gather(0, 0, 0)
    gather(1, 0, 1)

    def pairbody(q2, carry):
        for qq in range(2):
            qb = qq
            qn = 1 - qq
            q = q2 * 2 + qq
            for t in range(_SLAB):
                jj = q * _SLAB + t
                b = t % _RING
                wait_gather(t, qb, b)
                scatter(t, qb, b)
                if t == 2:
                    @pl.when(q + 1 < _NSLAB)
                    def _():
                        load_slab(q + 1, qn)
                if t == 13:
                    @pl.when(q + 1 < _NSLAB)
                    def _():
                        wait_slab(q + 1, qn)
                bn = (t + 2) % _RING

                @pl.when(jj >= 2)
                def _():
                    if t >= 2:
                        wait_scatter(t - 2, qb, bn)
                    else:
                        wait_scatter(t + _SLAB - 2, qn, bn)

                @pl.when(jj + 2 < _CPT_AGG)
                def _():
                    if t < _SLAB - 2:
                        gather(t + 2, qb, bn)
                    else:
                        gather(t + 2 - _SLAB, qn, bn)
        return carry

    lax.fori_loop(0, _NSLAB // 2, pairbody, 0)
    for t in (_SLAB - 2, _SLAB - 1):
        wait_scatter(t, (_NSLAB - 1) % 2, t % _RING)
    plsc.subcore_barrier()
    pltpu.sync_copy(acc_sh.at[pl.ds(s * _RPT, _RPT)],
                    out_hbm.at[c, pl.ds(s * _RPT, _RPT)])


def _sc_deg(dsts):
    fn = pl.kernel(
        _deg_body,
        out_type=jax.ShapeDtypeStruct((_NC, _HR, 16), _f32),
        mesh=_sc_mesh(),
        compiler_params=pltpu.CompilerParams(use_tc_tiling_on_sc=False,
                                             needs_layout_passes=False),
        scratch_types=[
            pltpu.VMEM((_CPT_DEG, _CW), jnp.int32),
            pltpu.VMEM((_HR, 16), _f32),
            pltpu.VMEM((_HR // _CW, _CW), jnp.int32),
            pltpu.VMEM_SHARED((_HR, 16), _f32),
        ],
    )
    return fn(dsts)


def _sc_agg(h, srcs, dsts):
    fn = pl.kernel(
        _agg_body,
        out_type=jax.ShapeDtypeStruct((_NC, _NPAD, _DH), _f32),
        mesh=_sc_mesh(),
        compiler_params=pltpu.CompilerParams(use_tc_tiling_on_sc=False),
        scratch_types=(
            [pltpu.VMEM((_SLAB, _CW), jnp.int32)] * 4
            + [pltpu.VMEM((_CW, _DH), _f32)] * _RING
            + [pltpu.VMEM_SHARED((_N, _DH), _f32)]
            + [pltpu.VMEM_SHARED((_NPAD, _DH), _f32)]
            + [pltpu.SemaphoreType.DMA] * (2 + 2 * _RING)
        ),
    )
    return fn(h, srcs, dsts)


def _row_spec():
    return pl.BlockSpec((_R, _D), lambda i: (i, 0))


def _half_spec():
    return pl.BlockSpec((_NC, _R, _DH), lambda i: (0, i, 0))


def _full_spec(shape):
    return pl.BlockSpec(shape, lambda i: tuple(0 for _ in shape))


def _split_store(out_ref, val):
    out_ref[0, :, :] = val[:, :_DH]
    out_ref[1, :, :] = val[:, _DH:]


def _cat(ref):
    return jnp.concatenate([ref[0], ref[1]], axis=-1)


def _mm_body(x_ref, w_ref, y_ref):
    y_ref[...] = jnp.dot(x_ref[...], w_ref[...],
                         preferred_element_type=_f32)


def _tc_mm(x, w):
    return pl.pallas_call(
        _mm_body,
        grid=(_G,),
        in_specs=[_row_spec(), _full_spec((_D, _D))],
        out_specs=_row_spec(),
        out_shape=jax.ShapeDtypeStruct((_N, _D), _f32),
    )(x, w)


def _scale1_body(deg_ref, y_ref, dinv_ref, h_ref):
    deg1 = deg_ref[0] + deg_ref[1] + 1.0
    dinv = jnp.broadcast_to(lax.rsqrt(deg1), (_R, _D))
    dinv_ref[...] = dinv
    _split_store(h_ref, dinv * y_ref[...])


def _tc_layer1(deg, y):
    return pl.pallas_call(
        _scale1_body,
        grid=(_G,),
        in_specs=[pl.BlockSpec((_NC, _R, 1), lambda i: (0, i, 0)),
                  _row_spec()],
        out_specs=[_row_spec(), _half_spec()],
        out_shape=[
            jax.ShapeDtypeStruct((_N, _D), _f32),
            jax.ShapeDtypeStruct((_NC, _N, _DH), _f32),
        ],
    )(deg, y)


def _layer_body(agg_ref, hp_ref, dinv_ref, b_ref, w_ref, out_ref):
    dinv = dinv_ref[...]
    a = dinv * (_cat(agg_ref) + _cat(hp_ref)) + b_ref[...]
    a = jnp.maximum(a, 0.0)
    y = jnp.dot(a, w_ref[...], preferred_element_type=_f32)
    _split_store(out_ref, dinv * y)


def _tc_layer(agg, hp, dinv, b, w):
    return pl.pallas_call(
        _layer_body,
        grid=(_G,),
        in_specs=[_half_spec(), _half_spec(), _row_spec(),
                  _full_spec((1, _D)), _full_spec((_D, _D))],
        out_specs=_half_spec(),
        out_shape=jax.ShapeDtypeStruct((_NC, _N, _DH), _f32),
    )(agg, hp, dinv, b, w)


def _final_body(agg_ref, hp_ref, dinv_ref, b_ref, out_ref):
    out_ref[...] = (dinv_ref[...] * (_cat(agg_ref) + _cat(hp_ref))
                    + b_ref[...])


def _tc_final(agg, hp, dinv, b):
    return pl.pallas_call(
        _final_body,
        grid=(_G,),
        in_specs=[_half_spec(), _half_spec(), _row_spec(),
                  _full_spec((1, _D))],
        out_specs=_row_spec(),
        out_shape=jax.ShapeDtypeStruct((_N, _D), _f32),
    )(agg, hp, dinv, b)


def kernel(x, edge_index, W1, b1, W2, b2, W3, b3):
    src = edge_index[0]
    dst = edge_index[1]
    pad = _EPAD - _E
    srcs = jnp.concatenate(
        [src, jnp.zeros((pad,), jnp.int32)]).reshape(_EPAD // _CW, _CW)
    dsts = jnp.concatenate(
        [dst, jnp.full((pad,), _N, jnp.int32)]).reshape(_EPAD // _CW, _CW)
    b1r = b1.reshape(1, _D)
    b2r = b2.reshape(1, _D)
    b3r = b3.reshape(1, _D)

    y1 = _tc_mm(x, W1)
    deg = _sc_deg(dsts)
    degc = deg.reshape(_NC, _NPAD, 1)[:, :_N]
    dinv, h1 = _tc_layer1(degc, y1)
    agg1 = _sc_agg(h1, srcs, dsts)
    h2 = _tc_layer(agg1[:, :_N], h1, dinv, b1r, W2)
    agg2 = _sc_agg(h2, srcs, dsts)
    h3 = _tc_layer(agg2[:, :_N], h2, dinv, b2r, W3)
    agg3 = _sc_agg(h3, srcs, dsts)
    return _tc_final(agg3[:, :_N], h3, dinv, b3r)
